# baseline (speedup 1.0000x reference)
import functools

import jax
import jax.numpy as jnp
import numpy as np
from jax import lax
from jax.experimental import pallas as pl
from jax.experimental.pallas import tpu as pltpu
from jax.experimental.pallas import tpu_sc as plsc


def _make_topk(B, C, H, W, h0, h1, w0, w1, NU=4):
    HW = H * W
    M = (h1 - h0) * (w1 - w0)
    wq = w1 - w0
    shift = W.bit_length() - 1
    CH = HW // NU

    def body(x_ref, o_idx, o_xt):
        xT = jnp.transpose(x_ref[0])
        o_xt[0] = xT
        n = jnp.sqrt(jnp.sum(xT * xT, axis=1, keepdims=True)) + 1e-8
        xn = (xT / n).astype(jnp.bfloat16)
        qn = jnp.concatenate(
            [xn[(h0 + r) * W + w0:(h0 + r) * W + w1] for r in range(h1 - h0)],
            axis=0)

        best_v = best_i = None
        for uc in range(NU):
            sim = lax.dot_general(
                qn, xn[uc * CH:(uc + 1) * CH],
                (((1,), (1,)), ((), ())),
                preferred_element_type=jnp.float32)
            ii = lax.broadcasted_iota(jnp.int32, sim.shape, 1) + uc * CH
            rr = lax.shift_right_logical(ii, shift)
            cc = jnp.bitwise_and(ii, W - 1)
            hole = (rr >= h0) & (rr < h1) & (cc >= w0) & (cc < w1)
            simm = jnp.where(hole, -jnp.inf, sim)
            mv = jnp.max(simm, axis=1)
            mi = jnp.argmax(simm, axis=1).astype(jnp.int32) + uc * CH
            if best_v is None:
                best_v, best_i = mv, mi
            else:
                take = mv > best_v
                best_i = jnp.where(take, mi, best_i)
                best_v = jnp.where(take, mv, best_v)
        o_idx[0, 0] = best_i + pl.program_id(0) * HW

    return pl.pallas_call(
        body,
        grid=(B,),
        in_specs=[pl.BlockSpec((1, C, HW), lambda b: (b, 0, 0))],
        out_specs=[pl.BlockSpec((1, 1, M), lambda b: (b, 0, 0)),
                   pl.BlockSpec((1, HW, C), lambda b: (b, 0, 0))],
        out_shape=[jax.ShapeDtypeStruct((B, 1, M), jnp.int32),
                   jax.ShapeDtypeStruct((B, HW, C), jnp.float32)],
    )


def _gather_blend(xt2, idxflat, qposg):
    R = idxflat.shape[0]
    C = xt2.shape[1]
    info = plsc.get_sparse_core_info()
    NC, NS = info.num_cores, info.num_subcores
    NW = NC * NS
    rpw = R // NW
    mesh = plsc.VectorSubcoreMesh(core_axis_name="c", subcore_axis_name="s")

    @functools.partial(
        pl.kernel, mesh=mesh,
        out_type=jax.ShapeDtypeStruct((R, C), jnp.float32),
        scratch_types=[
            pltpu.VMEM((rpw,), jnp.int32),
            pltpu.VMEM((rpw,), jnp.int32),
            pltpu.VMEM((rpw, C), jnp.float32),
            pltpu.VMEM((rpw, C), jnp.float32),
            pltpu.SemaphoreType.DMA,
            pltpu.SemaphoreType.DMA,
        ],
    )
    def sc_fn(xt_hbm, idx_hbm, qp_hbm, out_hbm,
              idx_v, qp_v, rows_v, q_v, sem0, sem1):
        wid = lax.axis_index("s") * NC + lax.axis_index("c")
        base = wid * rpw
        pltpu.sync_copy(idx_hbm.at[pl.ds(base, rpw)], idx_v)
        cp0 = pltpu.async_copy(xt_hbm.at[idx_v], rows_v, sem0)
        pltpu.sync_copy(qp_hbm.at[pl.ds(base, rpw)], qp_v)
        cp1 = pltpu.async_copy(xt_hbm.at[qp_v], q_v, sem1)
        cp0.wait()
        cp1.wait()

        def row(r, carry):
            for c in range(0, C, 16):
                s = pl.ds(c, 16)
                rows_v[r, s] = (rows_v[r, s] + q_v[r, s]) * 0.5
            return carry

        lax.fori_loop(0, rpw, row, 0)
        pltpu.sync_copy(rows_v, out_hbm.at[pl.ds(base, rpw)])

    return sc_fn(xt2, idxflat, qposg)


def kernel(input):
    x = input
    B, C, H, W = x.shape
    h0, h1 = H // 4, 3 * H // 4
    w0, w1 = W // 4, 3 * W // 4
    HW = H * W
    M = (h1 - h0) * (w1 - w0)

    x3 = x.reshape(B, C, HW)
    idx, xT = _make_topk(B, C, H, W, h0, h1, w0, w1)(x3)

    qposg = jnp.asarray(
        [b * HW + r * W + c
         for b in range(B) for r in range(h0, h1) for c in range(w0, w1)],
        dtype=jnp.int32)
    blended = _gather_blend(
        xT.reshape(B * HW, C), idx.reshape(B * M), qposg)

    patch = blended.reshape(B, h1 - h0, w1 - w0, C).transpose(0, 3, 1, 2)
    return x.at[:, :, h0:h1, w0:w1].set(patch)
